# Initial kernel scaffold; baseline (speedup 1.0000x reference)
import jax
import jax.numpy as jnp
from jax.experimental import pallas as pl


def kernel(loc_data, conf_data, loc_t, conf_t):
    raise NotImplementedError("write your pallas kernel here")


# speedup vs baseline: 26.2441x; 26.2441x over previous
import jax
import jax.numpy as jnp
from jax.experimental import pallas as pl
from jax.experimental.pallas import tpu as pltpu

_N = 8732
_NEGPOS = 3


def _tc_body(ld_ref, lt_ref, conf_ref, ct_ref, out_ref):
    pos = ct_ref[...] > 0
    posf = pos.astype(jnp.float32)
    B = posf.shape[0]

    sl1_box = jnp.zeros(posf.shape, jnp.float32)
    for c in range(4):
        d = ld_ref[c] - lt_ref[c]
        ad = jnp.abs(d)
        sl1_box = sl1_box + jnp.where(ad < 1.0, 0.5 * d * d, ad - 0.5)
    sl1_sum = jnp.sum(sl1_box * posf)
    n_pos_total = jnp.sum(posf)

    x = conf_ref[...]
    bce = jnp.maximum(x, 0.0) - x * posf + jnp.log1p(jnp.exp(-jnp.abs(x)))
    v = jnp.where(pos, 0.0, bce)

    num_pos = jnp.sum(posf, axis=1, keepdims=True)
    k = jnp.minimum(num_pos * float(_NEGPOS), float(_N))

    vb = jax.lax.bitcast_convert_type(v, jnp.int32)
    lo = jnp.zeros((B, 1), jnp.int32)
    hi = jnp.full((B, 1), 0x7F800000, jnp.int32)

    def body(_, carry):
        lo, hi = carry
        mid = lo + ((hi - lo) >> 1)
        cnt = jnp.sum((vb >= mid).astype(jnp.float32), axis=1, keepdims=True)
        ok = cnt >= k
        return jnp.where(ok, mid, lo), jnp.where(ok, hi, mid)

    lo, hi = jax.lax.fori_loop(0, 31, body, (lo, hi), unroll=True)
    t = jax.lax.bitcast_convert_type(lo, jnp.float32)

    gt = v > t
    gtf = gt.astype(jnp.float32)
    count_gt = jnp.sum(gtf, axis=1, keepdims=True)
    S_gt = jnp.sum(jnp.where(gt, v, 0.0), axis=1, keepdims=True)
    bce_pos_sum = jnp.sum(bce * posf, axis=1, keepdims=True)

    num_row = bce_pos_sum + S_gt + (k - count_gt) * t
    den_row = num_pos + count_gt + (k - count_gt) * (t > 0.0).astype(jnp.float32)

    Ntot = n_pos_total
    loss_l = sl1_sum / (4.0 * Ntot) / Ntot
    loss_c = jnp.sum(num_row) / jnp.sum(den_row) / Ntot
    col = jax.lax.broadcasted_iota(jnp.int32, (1, 8), 1)
    out_ref[...] = jnp.where(col == 0, loss_l, jnp.where(col == 1, loss_c, 0.0))


def kernel(loc_data, conf_data, loc_t, conf_t):
    ld = jnp.transpose(loc_data, (2, 0, 1))
    lt = jnp.transpose(loc_t, (2, 0, 1))
    conf = conf_data[..., 0]
    ct = conf_t.astype(jnp.int32)

    out = pl.pallas_call(
        _tc_body,
        out_shape=jax.ShapeDtypeStruct((1, 8), jnp.float32),
    )(ld, lt, conf, ct)
    return (out[0, 0], out[0, 1])
